# baseline (speedup 1.0000x reference)
import functools

import jax
import jax.numpy as jnp
import numpy as np
from jax import lax
from jax.experimental import pallas as pl
from jax.experimental.pallas import tpu as pltpu
from jax.experimental.pallas import tpu_sc as plsc

_NPOINT = 512
_NSAMPLE = 32
_RADIUS2 = np.float32(0.2 ** 2)
_TS = 512
_NW = 32
_CH = 128


def _fps_body(xt_ref, out_ref):
    B = xt_ref.shape[0]
    N = xt_ref.shape[2]
    R = N // 128
    xs, ys, zs, xall = [], [], [], []
    for b in range(B):
        x = xt_ref[b, 0, :].reshape(R, 128)
        y = xt_ref[b, 1, :].reshape(R, 128)
        z = xt_ref[b, 2, :].reshape(R, 128)
        xs.append(x); ys.append(y); zs.append(z)
        xall.append(jnp.concatenate([x, y, z], axis=1))
    rowiota = lax.broadcasted_iota(jnp.int32, (R, 128), 0)
    lane = lax.broadcasted_iota(jnp.int32, (1, 128), 1)
    rowc = lax.broadcasted_iota(jnp.int32, (R, 1), 0)
    oh0 = (lane == 0).astype(jnp.float32)
    oh1 = (lane == 1).astype(jnp.float32)
    oh2 = (lane == 2).astype(jnp.float32)

    def gather3(nxt, b):
        r = lax.shift_right_logical(nxt, 7)
        l = jnp.bitwise_and(nxt, 127)
        oneR = (rowc == r).astype(jnp.float32)
        oneL = (lane == l).astype(jnp.float32)
        tmp = jnp.sum(xall[b] * oneR, axis=0, keepdims=True)
        cx = jnp.sum(tmp[:, :128] * oneL, axis=1, keepdims=True)
        cy = jnp.sum(tmp[:, 128:256] * oneL, axis=1, keepdims=True)
        cz = jnp.sum(tmp[:, 256:] * oneL, axis=1, keepdims=True)
        return cx, cy, cz

    dist0, c0 = [], []
    for b in range(B):
        dist0.append(jnp.full((R, 128), 1e10, jnp.float32))
        c0.append((xs[b][0:1, 0:1], ys[b][0:1, 0:1], zs[b][0:1, 0:1]))

    def body(i, carry):
        new = []
        for b in range(B):
            dist, (cx, cy, cz) = carry[b]
            row = cx * oh0 + cy * oh1 + cz * oh2
            out_ref[b, pl.ds(i - 1, 1), :] = row
            dx = xs[b] - cx
            dy = ys[b] - cy
            dz = zs[b] - cz
            d = dx * dx + dy * dy
            d = d + dz * dz
            dist = jnp.minimum(dist, d)
            m_col = jnp.max(dist, axis=0, keepdims=True)
            r_l = jnp.min(jnp.where(dist == m_col, rowiota, R),
                          axis=0, keepdims=True)
            m = jnp.max(m_col, axis=1, keepdims=True)
            key = jnp.where(m_col == m, r_l * 128 + lane, N)
            nxt = jnp.min(key, axis=1, keepdims=True)
            new.append((dist, gather3(nxt, b)))
        return tuple(new)

    carry = lax.fori_loop(1, _NPOINT, body,
                          tuple((dist0[b], c0[b]) for b in range(B)))
    for b in range(B):
        _, (cx, cy, cz) = carry[b]
        row = cx * oh0 + cy * oh1 + cz * oh2
        out_ref[b, pl.ds(_NPOINT - 1, 1), :] = row


def _fps(xt):
    B, _, N = xt.shape
    return pl.pallas_call(
        _fps_body,
        out_shape=jax.ShapeDtypeStruct((B, _NPOINT, 128), jnp.float32),
    )(xt)


def _bq_body(xyz_ref, nt_ref, out_ref):
    N = xyz_ref.shape[1]
    W = N // 32
    x = xyz_ref[0, :, 0:1]
    y = xyz_ref[0, :, 1:2]
    z = xyz_ref[0, :, 2:3]
    cx = nt_ref[0, 0, :][None, :]
    cy = nt_ref[0, 1, :][None, :]
    cz = nt_ref[0, 2, :][None, :]
    dx = cx - x
    dy = cy - y
    dz = cz - z
    d2 = dx * dx + dy * dy
    d2 = d2 + dz * dz
    sub = lax.broadcasted_iota(jnp.int32, (N, 1), 0)
    pw = jnp.left_shift(jnp.int32(1), jnp.bitwise_and(sub, 31))
    bits = jnp.where(d2 < _RADIUS2, pw, 0)
    words = jnp.sum(bits.reshape(W, 32, _TS), axis=1)
    rowbase = lax.broadcasted_iota(jnp.int32, (W, _TS), 0) * 32
    rowi = lax.broadcasted_iota(jnp.int32, (W, _TS), 0)
    cols = []
    first = None
    for k in range(_NSAMPLE):
        b = jnp.bitwise_and(words, -words)
        e = lax.shift_right_logical(
            lax.bitcast_convert_type(b.astype(jnp.float32), jnp.int32), 23)
        bit = jnp.bitwise_and(e, 255) - 127
        cand = jnp.where(words != 0, rowbase + bit, N)
        v = jnp.min(cand, axis=0, keepdims=True)
        if k == 0:
            first = v
            cols.append(v)
        else:
            cols.append(jnp.where(v == N, first, v))
        wsel = rowi == lax.shift_right_logical(v, 5)
        pat = jnp.left_shift(jnp.int32(1), jnp.bitwise_and(v, 31))
        words = jnp.where(wsel, jnp.bitwise_xor(words, pat), words)
    out_ref[0] = jnp.concatenate(cols, axis=0)


def _ball_query(xyz, nt):
    B, N, _ = xyz.shape
    S = nt.shape[2]
    return pl.pallas_call(
        _bq_body,
        grid=(B, S // _TS),
        in_specs=[
            pl.BlockSpec((1, N, 3), lambda b, t: (b, 0, 0)),
            pl.BlockSpec((1, 3, _TS), lambda b, t: (b, 0, t)),
        ],
        out_specs=pl.BlockSpec((1, _NSAMPLE, _TS), lambda b, t: (b, 0, t)),
        out_shape=jax.ShapeDtypeStruct((B, _NSAMPLE, S), jnp.int32),
    )(xyz, nt)


def _sc_gather(tbl, idxg):
    total = idxg.shape[0]
    d = tbl.shape[1]
    per_w = total // _NW
    n_ch = per_w // _CH
    mesh = plsc.VectorSubcoreMesh(core_axis_name="c", subcore_axis_name="s")

    @functools.partial(
        pl.kernel,
        mesh=mesh,
        out_type=jax.ShapeDtypeStruct((total, d), jnp.float32),
        scratch_types=[
            pltpu.VMEM((_CH,), jnp.int32),
            pltpu.VMEM((_CH, d), jnp.float32),
            pltpu.SemaphoreType.DMA,
        ],
    )
    def gk(tbl_hbm, idx_hbm, out_hbm, idx_v, rows_v, sem):
        wid = lax.axis_index("s") * 2 + lax.axis_index("c")
        base = wid * per_w

        def step(j, carry):
            off = base + j * _CH
            pltpu.sync_copy(idx_hbm.at[pl.ds(off, _CH)], idx_v)
            pltpu.async_copy(tbl_hbm.at[idx_v], rows_v, sem).wait()
            pltpu.sync_copy(rows_v, out_hbm.at[pl.ds(off, _CH)])
            return carry

        lax.fori_loop(0, n_ch, step, 0)

    return gk(tbl, idxg)


def _mlp_body(g_ref, nx_ref, w1_ref, w1c_ref, b1_ref, w2_ref, b2_ref,
              w3_ref, b3_ref, out_ref):
    K = _NSAMPLE
    g = g_ref[0]
    h = jnp.dot(g, w1_ref[...], preferred_element_type=jnp.float32)
    c = nx_ref[0]
    ct = jnp.dot(c, w1c_ref[...], preferred_element_type=jnp.float32)
    h = h.reshape(_TS, K, h.shape[-1]) - ct[:, None, :]
    h = jnp.maximum(h + b1_ref[...][None], 0.0)
    h = h.reshape(_TS * K, h.shape[-1])
    h = jnp.maximum(
        jnp.dot(h, w2_ref[...], preferred_element_type=jnp.float32)
        + b2_ref[...], 0.0)
    h = jnp.maximum(
        jnp.dot(h, w3_ref[...], preferred_element_type=jnp.float32)
        + b3_ref[...], 0.0)
    out_ref[0] = jnp.max(h.reshape(_TS, K, h.shape[-1]), axis=1)


def _mlp(g, nxp, wbig, w1c, b1, w2, b2, w3, b3):
    B = g.shape[0]
    S = nxp.shape[1]
    D = g.shape[2]
    C3 = w3.shape[1]
    full = lambda shp: pl.BlockSpec(shp, lambda b, t: tuple(0 for _ in shp))
    return pl.pallas_call(
        _mlp_body,
        grid=(B, S // _TS),
        in_specs=[
            pl.BlockSpec((1, _TS * _NSAMPLE, D), lambda b, t: (b, t, 0)),
            pl.BlockSpec((1, _TS, 8), lambda b, t: (b, t, 0)),
            full(wbig.shape),
            full(w1c.shape),
            full(b1.shape),
            full(w2.shape),
            full(b2.shape),
            full(w3.shape),
            full(b3.shape),
        ],
        out_specs=pl.BlockSpec((1, _TS, C3), lambda b, t: (b, t, 0)),
        out_shape=jax.ShapeDtypeStruct((B, S, C3), jnp.float32),
    )(g, nxp, wbig, w1c, b1, w2, b2, w3, b3)


def kernel(xyz, features, W1, b1, W2, b2, W3, b3):
    B, N, _ = xyz.shape
    C = features.shape[1]
    S, K = _NPOINT, _NSAMPLE
    f32 = jnp.float32

    xt = jnp.transpose(xyz, (0, 2, 1))
    nx_pad = _fps(xt)
    new_xyz = nx_pad[:, :, :3]
    nt = jnp.transpose(new_xyz, (0, 2, 1))
    idx = jnp.transpose(_ball_query(xyz, nt), (0, 2, 1))

    pad = (-(C + 3)) % 128
    D = C + 3 + pad
    feats_t = jnp.transpose(features, (0, 2, 1))
    tbl = jnp.concatenate(
        [feats_t, xyz, jnp.zeros((B, N, pad), f32)], axis=-1
    ).reshape(B * N, D)
    idxg = (idx + (jnp.arange(B, dtype=jnp.int32) * N)[:, None, None]
            ).reshape(-1)
    g = _sc_gather(tbl, idxg).reshape(B, S * K, D)

    nxp = jnp.concatenate([new_xyz, jnp.zeros((B, S, 5), f32)], axis=-1)
    wbig = jnp.concatenate(
        [W1[3:], W1[:3], jnp.zeros((pad, W1.shape[1]), f32)], axis=0)
    w1c = jnp.concatenate([W1[:3], jnp.zeros((5, W1.shape[1]), f32)], axis=0)
    out = _mlp(g, nxp, wbig, w1c, b1.reshape(1, -1), W2, b2.reshape(1, -1),
               W3, b3.reshape(1, -1))
    new_features = jnp.transpose(out, (0, 2, 1))
    return (new_xyz, new_features)
